# Initial kernel scaffold; baseline (speedup 1.0000x reference)
import jax
import jax.numpy as jnp
from jax.experimental import pallas as pl


def kernel(x, original_weight, new_embedding):
    raise NotImplementedError("write your pallas kernel here")


# speedup vs baseline: 4.1965x; 4.1965x over previous
import functools

import jax
import jax.numpy as jnp
from jax import lax
from jax.experimental import pallas as pl
from jax.experimental.pallas import tpu as pltpu
from jax.experimental.pallas import tpu_sc as plsc

V = 100000
N_NEW = 200
D = 128

NC = 2
NS = 16
NW = NC * NS

CHUNK = 128


def _make_gather(total_rows: int):
    per_w = total_rows // NW
    n_chunks = per_w // CHUNK
    mesh = plsc.VectorSubcoreMesh(core_axis_name="c", subcore_axis_name="s")

    @functools.partial(
        pl.kernel,
        mesh=mesh,
        out_type=jax.ShapeDtypeStruct((total_rows, D), jnp.float32),
        scratch_types=[
            pltpu.VMEM((CHUNK,), jnp.int32),
            pltpu.VMEM((CHUNK, D), jnp.float32),
            pltpu.SemaphoreType.DMA,
        ],
    )
    def gather_kernel(idx_hbm, table_hbm, out_hbm, idx_v, rows_v, sem):
        wid = lax.axis_index("s") * NC + lax.axis_index("c")
        base = wid * per_w

        def body(i, carry):
            off = base + i * CHUNK
            pltpu.sync_copy(idx_hbm.at[pl.ds(off, CHUNK)], idx_v)
            pltpu.async_copy(table_hbm.at[idx_v], rows_v, sem).wait()
            pltpu.sync_copy(rows_v, out_hbm.at[pl.ds(off, CHUNK)])
            return carry

        lax.fori_loop(0, n_chunks, body, 0, unroll=False)

    return gather_kernel


def kernel(x, original_weight, new_embedding):
    table = jnp.concatenate([original_weight, new_embedding], axis=0)
    idx = x.reshape(-1).astype(jnp.int32)
    out = _make_gather(idx.shape[0])(idx, table)
    return out.reshape(x.shape + (D,))


# speedup vs baseline: 5.7714x; 1.3753x over previous
import functools

import jax
import jax.numpy as jnp
from jax import lax
from jax.experimental import pallas as pl
from jax.experimental.pallas import tpu as pltpu
from jax.experimental.pallas import tpu_sc as plsc

V = 100000
N_NEW = 200
D = 128

NC = 2
NS = 16
NW = NC * NS

CHUNK = 128


def _make_gather(total_rows: int):
    per_w = total_rows // NW
    n_chunks = per_w // CHUNK
    assert n_chunks % 2 == 0 and n_chunks >= 4
    mesh = plsc.VectorSubcoreMesh(core_axis_name="c", subcore_axis_name="s")

    @functools.partial(
        pl.kernel,
        mesh=mesh,
        out_type=jax.ShapeDtypeStruct((total_rows, D), jnp.float32),
        scratch_types=[
            pltpu.VMEM((2, CHUNK), jnp.int32),
            pltpu.VMEM((CHUNK, D), jnp.float32),
            pltpu.VMEM((CHUNK, D), jnp.float32),
            pltpu.SemaphoreType.DMA,
            pltpu.SemaphoreType.DMA,
            pltpu.SemaphoreType.DMA,
            pltpu.SemaphoreType.DMA,
        ],
    )
    def gather_kernel(idx_hbm, table_hbm, out_hbm, idx_v, rows0, rows1,
                      in0, in1, out0, out1):
        wid = lax.axis_index("s") * NC + lax.axis_index("c")
        base = wid * per_w
        rows = (rows0, rows1)
        sem_in = (in0, in1)
        sem_out = (out0, out1)

        def fire_gather(g, b):
            off = base + g * CHUNK
            pltpu.sync_copy(idx_hbm.at[pl.ds(off, CHUNK)], idx_v.at[b])
            pltpu.async_copy(table_hbm.at[idx_v.at[b]], rows[b], sem_in[b])

        def fire_out(g, b):
            off = base + g * CHUNK
            pltpu.async_copy(rows[b], out_hbm.at[pl.ds(off, CHUNK)],
                             sem_out[b])

        def wait_gather(b):
            pltpu.make_async_copy(table_hbm.at[idx_v.at[b]], rows[b],
                                  sem_in[b]).wait()

        def wait_out(g, b):
            off = base + g * CHUNK
            pltpu.make_async_copy(rows[b], out_hbm.at[pl.ds(off, CHUNK)],
                                  sem_out[b]).wait()

        fire_gather(0, 0)
        fire_gather(1, 1)
        wait_gather(0)
        fire_out(0, 0)

        def body(outer, carry):
            for b in range(2):
                g = 2 * outer + b
                wait_out(g - 2, b)
                fire_gather(g, b)
                wait_gather(1 - b)
                fire_out(g - 1, 1 - b)
            return carry

        lax.fori_loop(1, n_chunks // 2, body, 0, unroll=False)

        wait_gather(1)
        fire_out(n_chunks - 1, 1)
        wait_out(n_chunks - 2, 0)
        wait_out(n_chunks - 1, 1)

    return gather_kernel


def kernel(x, original_weight, new_embedding):
    table = jnp.concatenate([original_weight, new_embedding], axis=0)
    idx = x.reshape(-1).astype(jnp.int32)
    out = _make_gather(idx.shape[0])(idx, table)
    return out.reshape(x.shape + (D,))
